# Initial kernel scaffold; baseline (speedup 1.0000x reference)
import jax
import jax.numpy as jnp
from jax.experimental import pallas as pl


def kernel(params, node_ebd_ext, edge_ebd, h2, angle_ebd, nlist, nlist_mask, sw, a_nlist, a_nlist_mask, a_sw, edge_index, angle_index, edge_rbf, angle_rbf):
    raise NotImplementedError("write your pallas kernel here")


# speedup vs baseline: 2.1757x; 2.1757x over previous
import functools

import jax
import jax.numpy as jnp
import numpy as np
from jax.experimental import pallas as pl
from jax.experimental.pallas import tpu as pltpu

_NDIM = 128
_EDIM = 64
_ADIM = 32
_GH = 128


def _silu(x):
    return x * jax.nn.sigmoid(x)


def _dot(a, b):
    return jax.lax.dot_general(a, b, (((1,), (0,)), ((), ())),
                               preferred_element_type=jnp.float32)


def _full(shape):
    return pl.BlockSpec(shape, lambda i: (0,) * len(shape))


def _rows(b, d):
    return pl.BlockSpec((b, d), lambda i: (i, 0))


def _d1_body(ang_ref, ik_ref, ij_ref, asw_ref,
             wl_ref, gw_ref, gb_ref, uw_ref, ub_ref, ow_ref, ob_ref,
             amw_ref, amb_ref, ares_ref,
             scat_ref, angout_ref):
    ang = ang_ref[...]
    ik = ik_ref[...]
    ij = ij_ref[...]
    asw = asw_ref[...]
    gw = gw_ref[...]; uw = uw_ref[...]
    g = (_dot(ang, gw[0:_ADIM]) + _dot(ik, gw[_ADIM:_ADIM + _EDIM])
         + _dot(ij, gw[_ADIM + _EDIM:]) + gb_ref[...])
    u = (_dot(ang, uw[0:_ADIM]) + _dot(ik, uw[_ADIM:_ADIM + _EDIM])
         + _dot(ij, uw[_ADIM + _EDIM:]) + ub_ref[...])
    upd = _dot(_silu(g) * u, ow_ref[...]) + ob_ref[...]
    ex = jnp.exp(_dot(ang, wl_ref[...]) * asw)
    scat_ref[:, 0:_EDIM] = ex
    scat_ref[:, _EDIM:] = ex * upd * asw
    amw = amw_ref[...]
    a_upd = _silu(_dot(ang, amw[0:_ADIM]) + _dot(ik, amw[_ADIM:_ADIM + _EDIM])
                  + _dot(ij, amw[_ADIM + _EDIM:]) + amb_ref[...])
    angout_ref[...] = a_upd + ares_ref[...] * ang


def _d1(p, ang, ik, ij, asw, b):
    n = ang.shape[0]
    m = p['line_attn_edge_mlp']
    return pl.pallas_call(
        _d1_body,
        grid=(n // b,),
        in_specs=[_rows(b, _ADIM), _rows(b, _EDIM), _rows(b, _EDIM),
                  _rows(b, 1),
                  _full((_ADIM, _EDIM)),
                  _full((_ADIM + 2 * _EDIM, _GH)), _full((1, _GH)),
                  _full((_ADIM + 2 * _EDIM, _GH)), _full((1, _GH)),
                  _full((_GH, _EDIM)), _full((1, _EDIM)),
                  _full((_ADIM + 2 * _EDIM, _ADIM)), _full((1, _ADIM)),
                  _full((1, _ADIM))],
        out_specs=[_rows(b, 2 * _EDIM), _rows(b, _ADIM)],
        out_shape=[jax.ShapeDtypeStruct((n, 2 * _EDIM), jnp.float32),
                   jax.ShapeDtypeStruct((n, _ADIM), jnp.float32)],
        compiler_params=pltpu.CompilerParams(
            dimension_semantics=("arbitrary",)),
    )(ang, ik, ij, asw,
      p['line_attn_weight_linear']['w'],
      m['g']['w'], m['g']['b'][None], m['u']['w'], m['u']['b'][None],
      m['o']['w'], m['o']['b'][None],
      p['line_attn_angle_mlp']['w'], p['line_attn_angle_mlp']['b'][None],
      p['line_attn_angle_res'])


def _d2_body(n1_ref, n2_ref, s1_ref, eold_ref, sw_ref,
             gw_ref, gb_ref, uw_ref, ub_ref, ow_ref, ob_ref,
             wl_ref, res1_ref, res2_ref, inv_ref,
             scat_ref, e2_ref):
    den = s1_ref[:, 0:_EDIM]
    num = s1_ref[:, _EDIM:]
    e1 = num / (den + 1e-9) * inv_ref[0, 0] + res1_ref[...] * eold_ref[...]
    n1 = n1_ref[...]
    n2 = n2_ref[...]
    sw = sw_ref[...]
    gw = gw_ref[...]; uw = uw_ref[...]
    g = (_dot(n1, gw[0:_NDIM]) + _dot(n2, gw[_NDIM:2 * _NDIM])
         + _dot(e1, gw[2 * _NDIM:]) + gb_ref[...])
    u = (_dot(n1, uw[0:_NDIM]) + _dot(n2, uw[_NDIM:2 * _NDIM])
         + _dot(e1, uw[2 * _NDIM:]) + ub_ref[...])
    aeu = _dot(_silu(g) * u, ow_ref[...]) + ob_ref[...]
    ex = jnp.exp(_dot(e1, wl_ref[...]) * sw)
    scat_ref[:, 0:_EDIM] = ex
    scat_ref[:, _EDIM:] = ex * aeu * sw
    e2_ref[...] = aeu + res2_ref[...] * e1


def _d2(p, n1, n2, s1, eold, sw, b):
    n = n1.shape[0]
    m = p['atom_attn_edge_mlp']
    inv = jnp.full((1, 1), 1.0 / np.sqrt(0.8), jnp.float32)
    return pl.pallas_call(
        _d2_body,
        grid=(n // b,),
        in_specs=[_rows(b, _NDIM), _rows(b, _NDIM), _rows(b, 2 * _EDIM),
                  _rows(b, _EDIM), _rows(b, 1),
                  _full((2 * _NDIM + _EDIM, _GH)), _full((1, _GH)),
                  _full((2 * _NDIM + _EDIM, _GH)), _full((1, _GH)),
                  _full((_GH, _EDIM)), _full((1, _EDIM)),
                  _full((_EDIM, _EDIM)), _full((1, _EDIM)),
                  _full((1, _EDIM)), _full((1, 1))],
        out_specs=[_rows(b, 2 * _EDIM), _rows(b, _EDIM)],
        out_shape=[jax.ShapeDtypeStruct((n, 2 * _EDIM), jnp.float32),
                   jax.ShapeDtypeStruct((n, _EDIM), jnp.float32)],
        compiler_params=pltpu.CompilerParams(
            dimension_semantics=("arbitrary",)),
    )(n1, n2, s1, eold, sw,
      m['g']['w'], m['g']['b'][None], m['u']['w'], m['u']['b'][None],
      m['o']['w'], m['o']['b'][None],
      p['atom_attn_weight_linear']['w'],
      p['line_attn_edge_res'], p['atom_attn_edge_res'], inv)


def _d3_body(node_ref, s2_ref, gw_ref, gb_ref, uw_ref, ub_ref,
             ow_ref, ob_ref, res_ref, node1_ref):
    node = node_ref[...]
    den = s2_ref[:, 0:_EDIM]
    num = s2_ref[:, _EDIM:]
    agg = num / (den + 1e-9) * (1.0 / 3.2)
    gw = gw_ref[...]; uw = uw_ref[...]
    g = _dot(node, gw[0:_NDIM]) + _dot(agg, gw[_NDIM:]) + gb_ref[...]
    u = _dot(node, uw[0:_NDIM]) + _dot(agg, uw[_NDIM:]) + ub_ref[...]
    upd = _dot(_silu(g) * u, ow_ref[...]) + ob_ref[...]
    node1_ref[...] = upd + res_ref[...] * node


def _d3(p, node, s2, b):
    n = node.shape[0]
    m = p['atom_attn_node_mlp']
    return pl.pallas_call(
        _d3_body,
        grid=(n // b,),
        in_specs=[_rows(b, _NDIM), _rows(b, 2 * _EDIM),
                  _full((_NDIM + _EDIM, _GH)), _full((1, _GH)),
                  _full((_NDIM + _EDIM, _GH)), _full((1, _GH)),
                  _full((_GH, _NDIM)), _full((1, _NDIM)),
                  _full((1, _NDIM))],
        out_specs=[_rows(b, _NDIM)],
        out_shape=[jax.ShapeDtypeStruct((n, _NDIM), jnp.float32)],
        compiler_params=pltpu.CompilerParams(
            dimension_semantics=("arbitrary",)),
    )(node, s2,
      m['g']['w'], m['g']['b'][None], m['u']['w'], m['u']['b'][None],
      m['o']['w'], m['o']['b'][None], p['atom_attn_node_res'])[0]


def _d4_body(ng_ref, ang_ref, ik_ref, ij_ref, asw_ref, arbf_ref,
             gw_ref, gb_ref, uw_ref, ub_ref, ow_ref, ob_ref,
             envw_ref, effnw_ref, effnb_ref, res_ref,
             scat_ref, ang2_ref):
    ng = ng_ref[...]
    ang = ang_ref[...]
    ik = ik_ref[...]
    ij = ij_ref[...]
    asw = asw_ref[...]
    gw = gw_ref[...]; uw = uw_ref[...]
    c0, c1, c2 = _NDIM, _NDIM + _ADIM, _NDIM + _ADIM + _EDIM
    g = (_dot(ng, gw[0:c0]) + _dot(ang, gw[c0:c1]) + _dot(ik, gw[c1:c2])
         + _dot(ij, gw[c2:]) + gb_ref[...])
    u = (_dot(ng, uw[0:c0]) + _dot(ang, uw[c0:c1]) + _dot(ik, uw[c1:c2])
         + _dot(ij, uw[c2:]) + ub_ref[...])
    env = _dot(arbf_ref[...], envw_ref[...])
    lru = (_dot(_silu(g) * u, ow_ref[...]) + ob_ref[...]) * env
    scat_ref[...] = lru * asw
    ang2_ref[...] = _silu(_dot(lru, effnw_ref[...]) + effnb_ref[...]) \
        + res_ref[...] * ang


def _d4(p, ng, ang, ik, ij, asw, arbf, b):
    n = ang.shape[0]
    m = p['line_refine_mlp']
    din = _NDIM + _ADIM + 2 * _EDIM
    nab = arbf.shape[1]
    return pl.pallas_call(
        _d4_body,
        grid=(n // b,),
        in_specs=[_rows(b, _NDIM), _rows(b, _ADIM), _rows(b, _EDIM),
                  _rows(b, _EDIM), _rows(b, 1), _rows(b, nab),
                  _full((din, _GH)), _full((1, _GH)),
                  _full((din, _GH)), _full((1, _GH)),
                  _full((_GH, _EDIM)), _full((1, _EDIM)),
                  _full((nab, _EDIM)),
                  _full((_EDIM, _ADIM)), _full((1, _ADIM)),
                  _full((1, _ADIM))],
        out_specs=[_rows(b, _EDIM), _rows(b, _ADIM)],
        out_shape=[jax.ShapeDtypeStruct((n, _EDIM), jnp.float32),
                   jax.ShapeDtypeStruct((n, _ADIM), jnp.float32)],
        compiler_params=pltpu.CompilerParams(
            dimension_semantics=("arbitrary",)),
    )(ng, ang, ik, ij, asw, arbf,
      m['g']['w'], m['g']['b'][None], m['u']['w'], m['u']['b'][None],
      m['o']['w'], m['o']['b'][None],
      jnp.pad(p['line_refine_envelope']['w'], ((0, nab - 7), (0, 0))),
      p['line_refine_edge_ffn']['w'], p['line_refine_edge_ffn']['b'][None],
      p['line_refine_angle_res'])


def _d5_body(s3_ref, e2_ref, n1_ref, n2_ref, sw_ref, erbf_ref,
             nffnw_ref, nffnb_ref, resl_ref,
             gw_ref, gb_ref, uw_ref, ub_ref, ow_ref, ob_ref,
             envw_ref, effnw_ref, effnb_ref, rese_ref,
             scat_ref, e4_ref):
    agg = s3_ref[...] * (1.0 / 0.8)
    e3 = _silu(_dot(agg, nffnw_ref[...]) + nffnb_ref[...]) \
        + resl_ref[...] * e2_ref[...]
    n1 = n1_ref[...]
    n2 = n2_ref[...]
    sw = sw_ref[...]
    gw = gw_ref[...]; uw = uw_ref[...]
    g = (_dot(n1, gw[0:_NDIM]) + _dot(n2, gw[_NDIM:2 * _NDIM])
         + _dot(e3, gw[2 * _NDIM:]) + gb_ref[...])
    u = (_dot(n1, uw[0:_NDIM]) + _dot(n2, uw[_NDIM:2 * _NDIM])
         + _dot(e3, uw[2 * _NDIM:]) + ub_ref[...])
    env = _dot(erbf_ref[...], envw_ref[...])
    aru = (_dot(_silu(g) * u, ow_ref[...]) + ob_ref[...]) * env
    scat_ref[...] = aru * sw
    e4_ref[...] = _silu(_dot(aru, effnw_ref[...]) + effnb_ref[...]) \
        + rese_ref[...] * e3


def _d5(p, s3, e2, n1, n2, sw, erbf, b):
    n = e2.shape[0]
    m = p['atom_refine_mlp']
    neb = erbf.shape[1]
    return pl.pallas_call(
        _d5_body,
        grid=(n // b,),
        in_specs=[_rows(b, _EDIM), _rows(b, _EDIM), _rows(b, _NDIM),
                  _rows(b, _NDIM), _rows(b, 1), _rows(b, neb),
                  _full((_EDIM, _EDIM)), _full((1, _EDIM)),
                  _full((1, _EDIM)),
                  _full((2 * _NDIM + _EDIM, _GH)), _full((1, _GH)),
                  _full((2 * _NDIM + _EDIM, _GH)), _full((1, _GH)),
                  _full((_GH, _EDIM)), _full((1, _EDIM)),
                  _full((neb, _EDIM)),
                  _full((_EDIM, _EDIM)), _full((1, _EDIM)),
                  _full((1, _EDIM))],
        out_specs=[_rows(b, _EDIM), _rows(b, _EDIM)],
        out_shape=[jax.ShapeDtypeStruct((n, _EDIM), jnp.float32),
                   jax.ShapeDtypeStruct((n, _EDIM), jnp.float32)],
        compiler_params=pltpu.CompilerParams(
            dimension_semantics=("arbitrary",)),
    )(s3, e2, n1, n2, sw, erbf,
      p['line_refine_node_ffn']['w'], p['line_refine_node_ffn']['b'][None],
      p['line_refine_edge_res'],
      m['g']['w'], m['g']['b'][None], m['u']['w'], m['u']['b'][None],
      m['o']['w'], m['o']['b'][None],
      jnp.pad(p['atom_refine_envelope']['w'], ((0, neb - 7), (0, 0))),
      p['atom_refine_edge_ffn']['w'], p['atom_refine_edge_ffn']['b'][None],
      p['atom_refine_edge_res'])


def _d6_body(node1_ref, s4_ref, w_ref, b_ref, res_ref, node2_ref):
    agg = s4_ref[...] * (1.0 / 3.2)
    node2_ref[...] = _silu(_dot(agg, w_ref[...]) + b_ref[...]) \
        + res_ref[...] * node1_ref[...]


def _d6(p, node1, s4, b):
    n = node1.shape[0]
    return pl.pallas_call(
        _d6_body,
        grid=(n // b,),
        in_specs=[_rows(b, _NDIM), _rows(b, _EDIM),
                  _full((_EDIM, _NDIM)), _full((1, _NDIM)),
                  _full((1, _NDIM))],
        out_specs=[_rows(b, _NDIM)],
        out_shape=[jax.ShapeDtypeStruct((n, _NDIM), jnp.float32)],
        compiler_params=pltpu.CompilerParams(
            dimension_semantics=("arbitrary",)),
    )(node1, s4,
      p['atom_refine_node_ffn']['w'], p['atom_refine_node_ffn']['b'][None],
      p['atom_refine_node_res'])[0]


def _gather(table, idx):
    return jnp.take(table, idx, axis=0)


def _segsum(vals, seg, num):
    return jax.ops.segment_sum(vals, seg, num_segments=num)


def kernel(params, node_ebd_ext, edge_ebd, h2, angle_ebd, nlist, nlist_mask,
           sw, a_nlist, a_nlist_mask, a_sw, edge_index, angle_index,
           edge_rbf, angle_rbf):
    del h2, nlist, nlist_mask, a_nlist, a_nlist_mask
    p = params
    nb, nloc, _ = node_ebd_ext.shape
    n_edge = edge_ebd.shape[0]
    n_angle = angle_ebd.shape[0]
    n2e = edge_index[0]
    next2e = edge_index[1]
    n2a = angle_index[0]
    eij2a = angle_index[1]
    eik2a = angle_index[2]
    node_flat = node_ebd_ext.reshape(-1, _NDIM)
    asw = a_sw[:, None]
    swc = sw[:, None]
    be = 2000
    bn = 1000

    ik = _gather(edge_ebd, eik2a)
    ij = _gather(edge_ebd, eij2a)
    scat1, angle_1 = _d1(p, angle_ebd, ik, ij, asw, be)
    s1 = _segsum(scat1, eij2a, n_edge)

    ng1 = _gather(node_flat, n2e)
    ng2 = _gather(node_flat, next2e)
    scat2, e2 = _d2(p, ng1, ng2, s1, edge_ebd, swc, be)
    s2 = _segsum(scat2, n2e, nb * nloc)
    node_1 = _d3(p, node_flat, s2, bn)

    ik2 = _gather(e2, eik2a)
    ij2 = _gather(e2, eij2a)
    ng3 = _gather(node_1, n2a)
    arbf = jnp.pad(angle_rbf, ((0, 0), (0, 1)))
    scat3, angle_2 = _d4(p, ng3, angle_1, ik2, ij2, asw, arbf, be)
    s3 = _segsum(scat3, eij2a, n_edge)

    ng4 = _gather(node_1, n2e)
    erbf = jnp.pad(edge_rbf, ((0, 0), (0, 1)))
    scat4, e4 = _d5(p, s3, e2, ng4, ng2, swc, erbf, be)
    s4 = _segsum(scat4, n2e, nb * nloc)
    node_2 = _d6(p, node_1, s4, bn)

    return node_2.reshape(nb, nloc, _NDIM), e4, angle_2


# speedup vs baseline: 2.7296x; 1.2546x over previous
import functools

import jax
import jax.numpy as jnp
import numpy as np
from jax import lax
from jax.experimental import pallas as pl
from jax.experimental.pallas import tpu as pltpu
from jax.experimental.pallas import tpu_sc as plsc

_NDIM = 128
_EDIM = 64
_ADIM = 32
_GH = 128


def _silu(x):
    return x * jax.nn.sigmoid(x)


def _dot(a, b):
    return jax.lax.dot_general(a, b, (((1,), (0,)), ((), ())),
                               preferred_element_type=jnp.float32)


def _full(shape):
    return pl.BlockSpec(shape, lambda i: (0,) * len(shape))


def _rows(b, d):
    return pl.BlockSpec((b, d), lambda i: (i, 0))


def _rows_off(b, d, off):
    return pl.BlockSpec((b, d), lambda i: (i + off, 0))


def _d1_body(ang_ref, ik_ref, ij_ref, asw_ref,
             wl_ref, gw_ref, gb_ref, uw_ref, ub_ref, ow_ref, ob_ref,
             amw_ref, amb_ref, ares_ref,
             scat_ref, angout_ref):
    ang = ang_ref[...]
    ik = ik_ref[:, 0:_EDIM]
    ij = ij_ref[:, 0:_EDIM]
    asw = asw_ref[...]
    gw = gw_ref[...]; uw = uw_ref[...]
    g = (_dot(ang, gw[0:_ADIM]) + _dot(ik, gw[_ADIM:_ADIM + _EDIM])
         + _dot(ij, gw[_ADIM + _EDIM:]) + gb_ref[...])
    u = (_dot(ang, uw[0:_ADIM]) + _dot(ik, uw[_ADIM:_ADIM + _EDIM])
         + _dot(ij, uw[_ADIM + _EDIM:]) + ub_ref[...])
    upd = _dot(_silu(g) * u, ow_ref[...]) + ob_ref[...]
    ex = jnp.exp(_dot(ang, wl_ref[...]) * asw)
    scat_ref[:, 0:_EDIM] = ex
    scat_ref[:, _EDIM:] = ex * upd * asw
    amw = amw_ref[...]
    a_upd = _silu(_dot(ang, amw[0:_ADIM]) + _dot(ik, amw[_ADIM:_ADIM + _EDIM])
                  + _dot(ij, amw[_ADIM + _EDIM:]) + amb_ref[...])
    angout_ref[...] = a_upd + ares_ref[...] * ang


def _d1(p, ang, ikj, asw, b):
    n = ang.shape[0]
    off = n // b
    m = p['line_attn_edge_mlp']
    return pl.pallas_call(
        _d1_body,
        grid=(n // b,),
        in_specs=[_rows(b, _ADIM), _rows(b, 2 * _EDIM),
                  _rows_off(b, 2 * _EDIM, off), _rows(b, 1),
                  _full((_ADIM, _EDIM)),
                  _full((_ADIM + 2 * _EDIM, _GH)), _full((1, _GH)),
                  _full((_ADIM + 2 * _EDIM, _GH)), _full((1, _GH)),
                  _full((_GH, _EDIM)), _full((1, _EDIM)),
                  _full((_ADIM + 2 * _EDIM, _ADIM)), _full((1, _ADIM)),
                  _full((1, _ADIM))],
        out_specs=[_rows(b, 2 * _EDIM), _rows(b, _ADIM)],
        out_shape=[jax.ShapeDtypeStruct((n, 2 * _EDIM), jnp.float32),
                   jax.ShapeDtypeStruct((n, _ADIM), jnp.float32)],
        compiler_params=pltpu.CompilerParams(
            dimension_semantics=("arbitrary",)),
    )(ang, ikj, ikj, asw,
      p['line_attn_weight_linear']['w'],
      m['g']['w'], m['g']['b'][None], m['u']['w'], m['u']['b'][None],
      m['o']['w'], m['o']['b'][None],
      p['line_attn_angle_mlp']['w'], p['line_attn_angle_mlp']['b'][None],
      p['line_attn_angle_res'])


def _d2_body(n1_ref, n2_ref, s1_ref, eold_ref, sw_ref,
             gw_ref, gb_ref, uw_ref, ub_ref, ow_ref, ob_ref,
             wl_ref, res1_ref, res2_ref, inv_ref,
             scat_ref, e2_ref):
    den = s1_ref[:, 0:_EDIM]
    num = s1_ref[:, _EDIM:]
    e1 = num / (den + 1e-9) * inv_ref[0, 0] + res1_ref[...] * eold_ref[...]
    n1 = n1_ref[...]
    n2 = n2_ref[...]
    sw = sw_ref[...]
    gw = gw_ref[...]; uw = uw_ref[...]
    g = (_dot(n1, gw[0:_NDIM]) + _dot(n2, gw[_NDIM:2 * _NDIM])
         + _dot(e1, gw[2 * _NDIM:]) + gb_ref[...])
    u = (_dot(n1, uw[0:_NDIM]) + _dot(n2, uw[_NDIM:2 * _NDIM])
         + _dot(e1, uw[2 * _NDIM:]) + ub_ref[...])
    aeu = _dot(_silu(g) * u, ow_ref[...]) + ob_ref[...]
    ex = jnp.exp(_dot(e1, wl_ref[...]) * sw)
    scat_ref[:, 0:_EDIM] = ex
    scat_ref[:, _EDIM:] = ex * aeu * sw
    e2 = aeu + res2_ref[...] * e1
    e2_ref[:, 0:_EDIM] = e2
    e2_ref[:, _EDIM:] = e2


def _d2(p, ngg, s1, eold, sw, b):
    n = eold.shape[0]
    off = n // b
    m = p['atom_attn_edge_mlp']
    inv = jnp.full((1, 1), 1.0 / np.sqrt(0.8), jnp.float32)
    return pl.pallas_call(
        _d2_body,
        grid=(n // b,),
        in_specs=[_rows(b, _NDIM), _rows_off(b, _NDIM, off),
                  _rows(b, 2 * _EDIM),
                  _rows(b, _EDIM), _rows(b, 1),
                  _full((2 * _NDIM + _EDIM, _GH)), _full((1, _GH)),
                  _full((2 * _NDIM + _EDIM, _GH)), _full((1, _GH)),
                  _full((_GH, _EDIM)), _full((1, _EDIM)),
                  _full((_EDIM, _EDIM)), _full((1, _EDIM)),
                  _full((1, _EDIM)), _full((1, 1))],
        out_specs=[_rows(b, 2 * _EDIM), _rows(b, 2 * _EDIM)],
        out_shape=[jax.ShapeDtypeStruct((n, 2 * _EDIM), jnp.float32),
                   jax.ShapeDtypeStruct((n, 2 * _EDIM), jnp.float32)],
        compiler_params=pltpu.CompilerParams(
            dimension_semantics=("arbitrary",)),
    )(ngg, ngg, s1, eold, sw,
      m['g']['w'], m['g']['b'][None], m['u']['w'], m['u']['b'][None],
      m['o']['w'], m['o']['b'][None],
      p['atom_attn_weight_linear']['w'],
      p['line_attn_edge_res'], p['atom_attn_edge_res'], inv)


def _d3_body(node_ref, s2_ref, gw_ref, gb_ref, uw_ref, ub_ref,
             ow_ref, ob_ref, res_ref, node1_ref):
    node = node_ref[...]
    den = s2_ref[:, 0:_EDIM]
    num = s2_ref[:, _EDIM:]
    agg = num / (den + 1e-9) * (1.0 / 3.2)
    gw = gw_ref[...]; uw = uw_ref[...]
    g = _dot(node, gw[0:_NDIM]) + _dot(agg, gw[_NDIM:]) + gb_ref[...]
    u = _dot(node, uw[0:_NDIM]) + _dot(agg, uw[_NDIM:]) + ub_ref[...]
    upd = _dot(_silu(g) * u, ow_ref[...]) + ob_ref[...]
    node1_ref[...] = upd + res_ref[...] * node


def _d3(p, node, s2, b):
    n = node.shape[0]
    m = p['atom_attn_node_mlp']
    return pl.pallas_call(
        _d3_body,
        grid=(n // b,),
        in_specs=[_rows(b, _NDIM), _rows(b, 2 * _EDIM),
                  _full((_NDIM + _EDIM, _GH)), _full((1, _GH)),
                  _full((_NDIM + _EDIM, _GH)), _full((1, _GH)),
                  _full((_GH, _NDIM)), _full((1, _NDIM)),
                  _full((1, _NDIM))],
        out_specs=[_rows(b, _NDIM)],
        out_shape=[jax.ShapeDtypeStruct((n, _NDIM), jnp.float32)],
        compiler_params=pltpu.CompilerParams(
            dimension_semantics=("arbitrary",)),
    )(node, s2,
      m['g']['w'], m['g']['b'][None], m['u']['w'], m['u']['b'][None],
      m['o']['w'], m['o']['b'][None], p['atom_attn_node_res'])[0]


def _d4_body(ng_ref, ang_ref, ik_ref, ij_ref, asw_ref, arbf_ref,
             gw_ref, gb_ref, uw_ref, ub_ref, ow_ref, ob_ref,
             envw_ref, effnw_ref, effnb_ref, res_ref,
             scat_ref, ang2_ref):
    ng = ng_ref[...]
    ang = ang_ref[...]
    ik = ik_ref[:, 0:_EDIM]
    ij = ij_ref[:, 0:_EDIM]
    asw = asw_ref[...]
    gw = gw_ref[...]; uw = uw_ref[...]
    c0, c1, c2 = _NDIM, _NDIM + _ADIM, _NDIM + _ADIM + _EDIM
    g = (_dot(ng, gw[0:c0]) + _dot(ang, gw[c0:c1]) + _dot(ik, gw[c1:c2])
         + _dot(ij, gw[c2:]) + gb_ref[...])
    u = (_dot(ng, uw[0:c0]) + _dot(ang, uw[c0:c1]) + _dot(ik, uw[c1:c2])
         + _dot(ij, uw[c2:]) + ub_ref[...])
    env = _dot(arbf_ref[...], envw_ref[...])
    lru = (_dot(_silu(g) * u, ow_ref[...]) + ob_ref[...]) * env
    scat_ref[...] = lru * asw
    ang2_ref[...] = _silu(_dot(lru, effnw_ref[...]) + effnb_ref[...]) \
        + res_ref[...] * ang


def _d4(p, ng, ang, ikj, asw, arbf, b):
    n = ang.shape[0]
    off = n // b
    m = p['line_refine_mlp']
    din = _NDIM + _ADIM + 2 * _EDIM
    nab = arbf.shape[1]
    return pl.pallas_call(
        _d4_body,
        grid=(n // b,),
        in_specs=[_rows(b, _NDIM), _rows(b, _ADIM), _rows(b, 2 * _EDIM),
                  _rows_off(b, 2 * _EDIM, off), _rows(b, 1), _rows(b, nab),
                  _full((din, _GH)), _full((1, _GH)),
                  _full((din, _GH)), _full((1, _GH)),
                  _full((_GH, _EDIM)), _full((1, _EDIM)),
                  _full((nab, _EDIM)),
                  _full((_EDIM, _ADIM)), _full((1, _ADIM)),
                  _full((1, _ADIM))],
        out_specs=[_rows(b, _EDIM), _rows(b, _ADIM)],
        out_shape=[jax.ShapeDtypeStruct((n, _EDIM), jnp.float32),
                   jax.ShapeDtypeStruct((n, _ADIM), jnp.float32)],
        compiler_params=pltpu.CompilerParams(
            dimension_semantics=("arbitrary",)),
    )(ng, ang, ikj, ikj, asw, arbf,
      m['g']['w'], m['g']['b'][None], m['u']['w'], m['u']['b'][None],
      m['o']['w'], m['o']['b'][None],
      jnp.pad(p['line_refine_envelope']['w'], ((0, nab - 7), (0, 0))),
      p['line_refine_edge_ffn']['w'], p['line_refine_edge_ffn']['b'][None],
      p['line_refine_angle_res'])


def _d5_body(s3_ref, e2_ref, n1_ref, n2_ref, sw_ref, erbf_ref,
             nffnw_ref, nffnb_ref, resl_ref,
             gw_ref, gb_ref, uw_ref, ub_ref, ow_ref, ob_ref,
             envw_ref, effnw_ref, effnb_ref, rese_ref,
             scat_ref, e4_ref):
    agg = s3_ref[...] * (1.0 / 0.8)
    e3 = _silu(_dot(agg, nffnw_ref[...]) + nffnb_ref[...]) \
        + resl_ref[...] * e2_ref[:, 0:_EDIM]
    n1 = n1_ref[...]
    n2 = n2_ref[...]
    sw = sw_ref[...]
    gw = gw_ref[...]; uw = uw_ref[...]
    g = (_dot(n1, gw[0:_NDIM]) + _dot(n2, gw[_NDIM:2 * _NDIM])
         + _dot(e3, gw[2 * _NDIM:]) + gb_ref[...])
    u = (_dot(n1, uw[0:_NDIM]) + _dot(n2, uw[_NDIM:2 * _NDIM])
         + _dot(e3, uw[2 * _NDIM:]) + ub_ref[...])
    env = _dot(erbf_ref[...], envw_ref[...])
    aru = (_dot(_silu(g) * u, ow_ref[...]) + ob_ref[...]) * env
    scat_ref[...] = aru * sw
    e4_ref[...] = _silu(_dot(aru, effnw_ref[...]) + effnb_ref[...]) \
        + rese_ref[...] * e3


def _d5(p, s3, e2, n1, ngg, sw, erbf, b):
    n = e2.shape[0]
    off = n // b
    m = p['atom_refine_mlp']
    neb = erbf.shape[1]
    return pl.pallas_call(
        _d5_body,
        grid=(n // b,),
        in_specs=[_rows(b, _EDIM), _rows(b, 2 * _EDIM), _rows(b, _NDIM),
                  _rows_off(b, _NDIM, off), _rows(b, 1), _rows(b, neb),
                  _full((_EDIM, _EDIM)), _full((1, _EDIM)),
                  _full((1, _EDIM)),
                  _full((2 * _NDIM + _EDIM, _GH)), _full((1, _GH)),
                  _full((2 * _NDIM + _EDIM, _GH)), _full((1, _GH)),
                  _full((_GH, _EDIM)), _full((1, _EDIM)),
                  _full((neb, _EDIM)),
                  _full((_EDIM, _EDIM)), _full((1, _EDIM)),
                  _full((1, _EDIM))],
        out_specs=[_rows(b, _EDIM), _rows(b, _EDIM)],
        out_shape=[jax.ShapeDtypeStruct((n, _EDIM), jnp.float32),
                   jax.ShapeDtypeStruct((n, _EDIM), jnp.float32)],
        compiler_params=pltpu.CompilerParams(
            dimension_semantics=("arbitrary",)),
    )(s3, e2, n1, ngg, sw, erbf,
      p['line_refine_node_ffn']['w'], p['line_refine_node_ffn']['b'][None],
      p['line_refine_edge_res'],
      m['g']['w'], m['g']['b'][None], m['u']['w'], m['u']['b'][None],
      m['o']['w'], m['o']['b'][None],
      jnp.pad(p['atom_refine_envelope']['w'], ((0, neb - 7), (0, 0))),
      p['atom_refine_edge_ffn']['w'], p['atom_refine_edge_ffn']['b'][None],
      p['atom_refine_edge_res'])


def _d6_body(node1_ref, s4_ref, w_ref, b_ref, res_ref, node2_ref):
    agg = s4_ref[...] * (1.0 / 3.2)
    node2_ref[...] = _silu(_dot(agg, w_ref[...]) + b_ref[...]) \
        + res_ref[...] * node1_ref[...]


def _d6(p, node1, s4, b):
    n = node1.shape[0]
    return pl.pallas_call(
        _d6_body,
        grid=(n // b,),
        in_specs=[_rows(b, _NDIM), _rows(b, _EDIM),
                  _full((_EDIM, _NDIM)), _full((1, _NDIM)),
                  _full((1, _NDIM))],
        out_specs=[_rows(b, _NDIM)],
        out_shape=[jax.ShapeDtypeStruct((n, _NDIM), jnp.float32)],
        compiler_params=pltpu.CompilerParams(
            dimension_semantics=("arbitrary",)),
    )(node1, s4,
      p['atom_refine_node_ffn']['w'], p['atom_refine_node_ffn']['b'][None],
      p['atom_refine_node_res'])[0]


_NW = 32
_GCHUNK = 80


@functools.partial(jax.jit, static_argnames=("d",))
def _sc_gather(table, idx, d):
    b = idx.shape[0]
    per_w = b // _NW
    nchunk = per_w // _GCHUNK
    mesh = plsc.VectorSubcoreMesh(core_axis_name="c", subcore_axis_name="s")

    @functools.partial(
        pl.kernel, mesh=mesh,
        out_type=jax.ShapeDtypeStruct((b, d), jnp.float32),
        scratch_types=[
            pltpu.VMEM((_GCHUNK,), jnp.int32),
            pltpu.VMEM((_GCHUNK, d), jnp.float32),
            pltpu.VMEM((_GCHUNK,), jnp.int32),
            pltpu.VMEM((_GCHUNK, d), jnp.float32),
            pltpu.SemaphoreType.DMA,
            pltpu.SemaphoreType.DMA,
        ],
    )
    def k(table_hbm, idx_hbm, out_hbm, idx0, rows0, idx1, rows1, s0, s1):
        wid = lax.axis_index("s") * 2 + lax.axis_index("c")
        base = wid * per_w

        def issue(j, idx_v, rows_v, sem):
            pltpu.sync_copy(idx_hbm.at[pl.ds(base + j * _GCHUNK, _GCHUNK)],
                            idx_v)
            return pltpu.async_copy(table_hbm.at[idx_v], rows_v, sem)

        def drain(j, rows_v, cp):
            cp.wait()
            pltpu.sync_copy(rows_v,
                            out_hbm.at[pl.ds(base + j * _GCHUNK, _GCHUNK)])

        def body(i, _):
            j = i * 2
            cp0 = issue(j, idx0, rows0, s0)
            cp1 = issue(j + 1, idx1, rows1, s1)
            drain(j, rows0, cp0)
            drain(j + 1, rows1, cp1)
            return 0

        lax.fori_loop(0, nchunk // 2, body, 0)
        if nchunk % 2:
            drain(nchunk - 1, rows0, issue(nchunk - 1, idx0, rows0, s0))

    return k(table, idx)


def _gather(table, idx):
    return _sc_gather(table, idx, table.shape[1])


def _segsum(vals, seg, num):
    return jax.ops.segment_sum(vals, seg, num_segments=num)


def kernel(params, node_ebd_ext, edge_ebd, h2, angle_ebd, nlist, nlist_mask,
           sw, a_nlist, a_nlist_mask, a_sw, edge_index, angle_index,
           edge_rbf, angle_rbf):
    del h2, nlist, nlist_mask, a_nlist, a_nlist_mask
    p = params
    nb, nloc, _ = node_ebd_ext.shape
    n_edge = edge_ebd.shape[0]
    n_angle = angle_ebd.shape[0]
    n2e = edge_index[0]
    next2e = edge_index[1]
    n2a = angle_index[0]
    eij2a = angle_index[1]
    eik2a = angle_index[2]
    node_flat = node_ebd_ext.reshape(-1, _NDIM)
    asw = a_sw[:, None]
    swc = sw[:, None]
    be = 2000
    bn = 1000

    ikij = jnp.concatenate([eik2a, eij2a]).astype(jnp.int32)
    nene = jnp.concatenate([n2e, next2e]).astype(jnp.int32)

    edge_dup = jnp.concatenate([edge_ebd, edge_ebd], axis=1)
    ikj = _gather(edge_dup, ikij)
    scat1, angle_1 = _d1(p, angle_ebd, ikj, asw, be)
    s1 = _segsum(scat1, eij2a, n_edge)

    ngg = _gather(node_flat, nene)
    scat2, e2dup = _d2(p, ngg, s1, edge_ebd, swc, be)
    s2 = _segsum(scat2, n2e, nb * nloc)
    node_1 = _d3(p, node_flat, s2, bn)

    ikj2 = _gather(e2dup, ikij)
    ng3 = _gather(node_1, n2a.astype(jnp.int32))
    arbf = jnp.pad(angle_rbf, ((0, 0), (0, 1)))
    scat3, angle_2 = _d4(p, ng3, angle_1, ikj2, asw, arbf, be)
    s3 = _segsum(scat3, eij2a, n_edge)

    ng4 = _gather(node_1, n2e.astype(jnp.int32))
    erbf = jnp.pad(edge_rbf, ((0, 0), (0, 1)))
    scat4, e4 = _d5(p, s3, e2dup, ng4, ngg, swc, erbf, be)
    s4 = _segsum(scat4, n2e, nb * nloc)
    node_2 = _d6(p, node_1, s4, bn)

    return node_2.reshape(nb, nloc, _NDIM), e4, angle_2


# speedup vs baseline: 2.9506x; 1.0810x over previous
import functools

import jax
import jax.numpy as jnp
import numpy as np
from jax import lax
from jax.experimental import pallas as pl
from jax.experimental.pallas import tpu as pltpu
from jax.experimental.pallas import tpu_sc as plsc

_NDIM = 128
_EDIM = 64
_ADIM = 32
_GH = 128


def _silu(x):
    return x * jax.nn.sigmoid(x)


def _dot(a, b):
    return jax.lax.dot_general(a, b, (((1,), (0,)), ((), ())),
                               preferred_element_type=jnp.float32)


def _full(shape):
    return pl.BlockSpec(shape, lambda i: (0,) * len(shape))


def _rows(b, d):
    return pl.BlockSpec((b, d), lambda i: (i, 0))


def _rows_off(b, d, off):
    return pl.BlockSpec((b, d), lambda i: (i + off, 0))


def _d1_body(ang_ref, ik_ref, ij_ref, asw_ref,
             wl_ref, gw_ref, gb_ref, uw_ref, ub_ref, ow_ref, ob_ref,
             amw_ref, amb_ref, ares_ref,
             scat_ref, angout_ref):
    ang = ang_ref[...]
    ik = ik_ref[:, 0:_EDIM]
    ij = ij_ref[:, 0:_EDIM]
    asw = asw_ref[...]
    gw = gw_ref[...]; uw = uw_ref[...]
    g = (_dot(ang, gw[0:_ADIM]) + _dot(ik, gw[_ADIM:_ADIM + _EDIM])
         + _dot(ij, gw[_ADIM + _EDIM:]) + gb_ref[...])
    u = (_dot(ang, uw[0:_ADIM]) + _dot(ik, uw[_ADIM:_ADIM + _EDIM])
         + _dot(ij, uw[_ADIM + _EDIM:]) + ub_ref[...])
    upd = _dot(_silu(g) * u, ow_ref[...]) + ob_ref[...]
    ex = jnp.exp(_dot(ang, wl_ref[...]) * asw)
    scat_ref[:, 0:_EDIM] = ex
    scat_ref[:, _EDIM:] = ex * upd * asw
    amw = amw_ref[...]
    a_upd = _silu(_dot(ang, amw[0:_ADIM]) + _dot(ik, amw[_ADIM:_ADIM + _EDIM])
                  + _dot(ij, amw[_ADIM + _EDIM:]) + amb_ref[...])
    angout_ref[...] = a_upd + ares_ref[...] * ang


def _d1(p, ang, ikj, asw, b):
    n = ang.shape[0]
    off = n // b
    m = p['line_attn_edge_mlp']
    return pl.pallas_call(
        _d1_body,
        grid=(n // b,),
        in_specs=[_rows(b, _ADIM), _rows(b, 2 * _EDIM),
                  _rows_off(b, 2 * _EDIM, off), _rows(b, 1),
                  _full((_ADIM, _EDIM)),
                  _full((_ADIM + 2 * _EDIM, _GH)), _full((1, _GH)),
                  _full((_ADIM + 2 * _EDIM, _GH)), _full((1, _GH)),
                  _full((_GH, _EDIM)), _full((1, _EDIM)),
                  _full((_ADIM + 2 * _EDIM, _ADIM)), _full((1, _ADIM)),
                  _full((1, _ADIM))],
        out_specs=[_rows(b, 2 * _EDIM), _rows(b, _ADIM)],
        out_shape=[jax.ShapeDtypeStruct((n, 2 * _EDIM), jnp.float32),
                   jax.ShapeDtypeStruct((n, _ADIM), jnp.float32)],
        compiler_params=pltpu.CompilerParams(
            dimension_semantics=("arbitrary",)),
    )(ang, ikj, ikj, asw,
      p['line_attn_weight_linear']['w'],
      m['g']['w'], m['g']['b'][None], m['u']['w'], m['u']['b'][None],
      m['o']['w'], m['o']['b'][None],
      p['line_attn_angle_mlp']['w'], p['line_attn_angle_mlp']['b'][None],
      p['line_attn_angle_res'])


def _d2_body(n1_ref, n2_ref, s1_ref, eold_ref, sw_ref,
             gw_ref, gb_ref, uw_ref, ub_ref, ow_ref, ob_ref,
             wl_ref, res1_ref, res2_ref, inv_ref,
             scat_ref, e2_ref):
    den = s1_ref[:, 0:_EDIM]
    num = s1_ref[:, _EDIM:]
    e1 = num / (den + 1e-9) * inv_ref[0, 0] + res1_ref[...] * eold_ref[...]
    n1 = n1_ref[...]
    n2 = n2_ref[...]
    sw = sw_ref[...]
    gw = gw_ref[...]; uw = uw_ref[...]
    g = (_dot(n1, gw[0:_NDIM]) + _dot(n2, gw[_NDIM:2 * _NDIM])
         + _dot(e1, gw[2 * _NDIM:]) + gb_ref[...])
    u = (_dot(n1, uw[0:_NDIM]) + _dot(n2, uw[_NDIM:2 * _NDIM])
         + _dot(e1, uw[2 * _NDIM:]) + ub_ref[...])
    aeu = _dot(_silu(g) * u, ow_ref[...]) + ob_ref[...]
    ex = jnp.exp(_dot(e1, wl_ref[...]) * sw)
    scat_ref[:, 0:_EDIM] = ex
    scat_ref[:, _EDIM:] = ex * aeu * sw
    e2 = aeu + res2_ref[...] * e1
    e2_ref[:, 0:_EDIM] = e2
    e2_ref[:, _EDIM:] = e2


def _d2(p, ngg, s1, eold, sw, b):
    n = eold.shape[0]
    off = n // b
    m = p['atom_attn_edge_mlp']
    inv = jnp.full((1, 1), 1.0 / np.sqrt(0.8), jnp.float32)
    return pl.pallas_call(
        _d2_body,
        grid=(n // b,),
        in_specs=[_rows(b, _NDIM), _rows_off(b, _NDIM, off),
                  _rows(b, 2 * _EDIM),
                  _rows(b, _EDIM), _rows(b, 1),
                  _full((2 * _NDIM + _EDIM, _GH)), _full((1, _GH)),
                  _full((2 * _NDIM + _EDIM, _GH)), _full((1, _GH)),
                  _full((_GH, _EDIM)), _full((1, _EDIM)),
                  _full((_EDIM, _EDIM)), _full((1, _EDIM)),
                  _full((1, _EDIM)), _full((1, 1))],
        out_specs=[_rows(b, 2 * _EDIM), _rows(b, 2 * _EDIM)],
        out_shape=[jax.ShapeDtypeStruct((n, 2 * _EDIM), jnp.float32),
                   jax.ShapeDtypeStruct((n, 2 * _EDIM), jnp.float32)],
        compiler_params=pltpu.CompilerParams(
            dimension_semantics=("arbitrary",)),
    )(ngg, ngg, s1, eold, sw,
      m['g']['w'], m['g']['b'][None], m['u']['w'], m['u']['b'][None],
      m['o']['w'], m['o']['b'][None],
      p['atom_attn_weight_linear']['w'],
      p['line_attn_edge_res'], p['atom_attn_edge_res'], inv)


def _d3_body(node_ref, s2_ref, gw_ref, gb_ref, uw_ref, ub_ref,
             ow_ref, ob_ref, res_ref, node1_ref):
    node = node_ref[...]
    agg = s2_ref[:, _EDIM:] / (s2_ref[:, 0:_EDIM] + 1e-9) * (1.0 / 3.2)
    gw = gw_ref[...]; uw = uw_ref[...]
    g = _dot(node, gw[0:_NDIM]) + _dot(agg, gw[_NDIM:]) + gb_ref[...]
    u = _dot(node, uw[0:_NDIM]) + _dot(agg, uw[_NDIM:]) + ub_ref[...]
    upd = _dot(_silu(g) * u, ow_ref[...]) + ob_ref[...]
    node1_ref[...] = upd + res_ref[...] * node


def _d3(p, node, s2, b):
    n = node.shape[0]
    m = p['atom_attn_node_mlp']
    return pl.pallas_call(
        _d3_body,
        grid=(n // b,),
        in_specs=[_rows(b, _NDIM), _rows(b, 2 * _EDIM),
                  _full((_NDIM + _EDIM, _GH)), _full((1, _GH)),
                  _full((_NDIM + _EDIM, _GH)), _full((1, _GH)),
                  _full((_GH, _NDIM)), _full((1, _NDIM)),
                  _full((1, _NDIM))],
        out_specs=[_rows(b, _NDIM)],
        out_shape=[jax.ShapeDtypeStruct((n, _NDIM), jnp.float32)],
        compiler_params=pltpu.CompilerParams(
            dimension_semantics=("arbitrary",)),
    )(node, s2,
      m['g']['w'], m['g']['b'][None], m['u']['w'], m['u']['b'][None],
      m['o']['w'], m['o']['b'][None], p['atom_attn_node_res'])[0]


def _d4_body(ng_ref, ang_ref, ik_ref, ij_ref, asw_ref, arbf_ref,
             gw_ref, gb_ref, uw_ref, ub_ref, ow_ref, ob_ref,
             envw_ref, effnw_ref, effnb_ref, res_ref,
             scat_ref, ang2_ref):
    ng = ng_ref[...]
    ang = ang_ref[...]
    ik = ik_ref[:, 0:_EDIM]
    ij = ij_ref[:, 0:_EDIM]
    asw = asw_ref[...]
    gw = gw_ref[...]; uw = uw_ref[...]
    c0, c1, c2 = _NDIM, _NDIM + _ADIM, _NDIM + _ADIM + _EDIM
    g = (_dot(ng, gw[0:c0]) + _dot(ang, gw[c0:c1]) + _dot(ik, gw[c1:c2])
         + _dot(ij, gw[c2:]) + gb_ref[...])
    u = (_dot(ng, uw[0:c0]) + _dot(ang, uw[c0:c1]) + _dot(ik, uw[c1:c2])
         + _dot(ij, uw[c2:]) + ub_ref[...])
    env = _dot(arbf_ref[...], envw_ref[...])
    lru = (_dot(_silu(g) * u, ow_ref[...]) + ob_ref[...]) * env
    sc3 = lru * asw
    scat_ref[:, 0:_EDIM] = sc3
    scat_ref[:, _EDIM:] = sc3
    ang2_ref[...] = _silu(_dot(lru, effnw_ref[...]) + effnb_ref[...]) \
        + res_ref[...] * ang


def _d4(p, ng, ang, ikj, asw, arbf, b):
    n = ang.shape[0]
    off = n // b
    m = p['line_refine_mlp']
    din = _NDIM + _ADIM + 2 * _EDIM
    nab = arbf.shape[1]
    return pl.pallas_call(
        _d4_body,
        grid=(n // b,),
        in_specs=[_rows(b, _NDIM), _rows(b, _ADIM), _rows(b, 2 * _EDIM),
                  _rows_off(b, 2 * _EDIM, off), _rows(b, 1), _rows(b, nab),
                  _full((din, _GH)), _full((1, _GH)),
                  _full((din, _GH)), _full((1, _GH)),
                  _full((_GH, _EDIM)), _full((1, _EDIM)),
                  _full((nab, _EDIM)),
                  _full((_EDIM, _ADIM)), _full((1, _ADIM)),
                  _full((1, _ADIM))],
        out_specs=[_rows(b, 2 * _EDIM), _rows(b, _ADIM)],
        out_shape=[jax.ShapeDtypeStruct((n, 2 * _EDIM), jnp.float32),
                   jax.ShapeDtypeStruct((n, _ADIM), jnp.float32)],
        compiler_params=pltpu.CompilerParams(
            dimension_semantics=("arbitrary",)),
    )(ng, ang, ikj, ikj, asw, arbf,
      m['g']['w'], m['g']['b'][None], m['u']['w'], m['u']['b'][None],
      m['o']['w'], m['o']['b'][None],
      jnp.pad(p['line_refine_envelope']['w'], ((0, nab - 7), (0, 0))),
      p['line_refine_edge_ffn']['w'], p['line_refine_edge_ffn']['b'][None],
      p['line_refine_angle_res'])


def _d5_body(s3_ref, e2_ref, n1_ref, n2_ref, sw_ref, erbf_ref,
             nffnw_ref, nffnb_ref, resl_ref,
             gw_ref, gb_ref, uw_ref, ub_ref, ow_ref, ob_ref,
             envw_ref, effnw_ref, effnb_ref, rese_ref,
             scat_ref, e4_ref):
    agg = s3_ref[:, 0:_EDIM] * (1.0 / 0.8)
    e3 = _silu(_dot(agg, nffnw_ref[...]) + nffnb_ref[...]) \
        + resl_ref[...] * e2_ref[:, 0:_EDIM]
    n1 = n1_ref[...]
    n2 = n2_ref[...]
    sw = sw_ref[...]
    gw = gw_ref[...]; uw = uw_ref[...]
    g = (_dot(n1, gw[0:_NDIM]) + _dot(n2, gw[_NDIM:2 * _NDIM])
         + _dot(e3, gw[2 * _NDIM:]) + gb_ref[...])
    u = (_dot(n1, uw[0:_NDIM]) + _dot(n2, uw[_NDIM:2 * _NDIM])
         + _dot(e3, uw[2 * _NDIM:]) + ub_ref[...])
    env = _dot(erbf_ref[...], envw_ref[...])
    aru = (_dot(_silu(g) * u, ow_ref[...]) + ob_ref[...]) * env
    sc4 = aru * sw
    scat_ref[:, 0:_EDIM] = sc4
    scat_ref[:, _EDIM:] = sc4
    e4_ref[...] = _silu(_dot(aru, effnw_ref[...]) + effnb_ref[...]) \
        + rese_ref[...] * e3


def _d5(p, s3, e2, n1, ngg, sw, erbf, b):
    n = e2.shape[0]
    off = n // b
    m = p['atom_refine_mlp']
    neb = erbf.shape[1]
    return pl.pallas_call(
        _d5_body,
        grid=(n // b,),
        in_specs=[_rows(b, 2 * _EDIM), _rows(b, 2 * _EDIM), _rows(b, _NDIM),
                  _rows_off(b, _NDIM, off), _rows(b, 1), _rows(b, neb),
                  _full((_EDIM, _EDIM)), _full((1, _EDIM)),
                  _full((1, _EDIM)),
                  _full((2 * _NDIM + _EDIM, _GH)), _full((1, _GH)),
                  _full((2 * _NDIM + _EDIM, _GH)), _full((1, _GH)),
                  _full((_GH, _EDIM)), _full((1, _EDIM)),
                  _full((neb, _EDIM)),
                  _full((_EDIM, _EDIM)), _full((1, _EDIM)),
                  _full((1, _EDIM))],
        out_specs=[_rows(b, 2 * _EDIM), _rows(b, _EDIM)],
        out_shape=[jax.ShapeDtypeStruct((n, 2 * _EDIM), jnp.float32),
                   jax.ShapeDtypeStruct((n, _EDIM), jnp.float32)],
        compiler_params=pltpu.CompilerParams(
            dimension_semantics=("arbitrary",)),
    )(s3, e2, n1, ngg, sw, erbf,
      p['line_refine_node_ffn']['w'], p['line_refine_node_ffn']['b'][None],
      p['line_refine_edge_res'],
      m['g']['w'], m['g']['b'][None], m['u']['w'], m['u']['b'][None],
      m['o']['w'], m['o']['b'][None],
      jnp.pad(p['atom_refine_envelope']['w'], ((0, neb - 7), (0, 0))),
      p['atom_refine_edge_ffn']['w'], p['atom_refine_edge_ffn']['b'][None],
      p['atom_refine_edge_res'])


def _d6_body(node1_ref, s4_ref, w_ref, b_ref, res_ref, node2_ref):
    agg = s4_ref[:, 0:_EDIM] * (1.0 / 3.2)
    node2_ref[...] = _silu(_dot(agg, w_ref[...]) + b_ref[...]) \
        + res_ref[...] * node1_ref[...]


def _d6(p, node1, s4, b):
    n = node1.shape[0]
    return pl.pallas_call(
        _d6_body,
        grid=(n // b,),
        in_specs=[_rows(b, _NDIM), _rows(b, 2 * _EDIM),
                  _full((_EDIM, _NDIM)), _full((1, _NDIM)),
                  _full((1, _NDIM))],
        out_specs=[_rows(b, _NDIM)],
        out_shape=[jax.ShapeDtypeStruct((n, _NDIM), jnp.float32)],
        compiler_params=pltpu.CompilerParams(
            dimension_semantics=("arbitrary",)),
    )(node1, s4,
      p['atom_refine_node_ffn']['w'], p['atom_refine_node_ffn']['b'][None],
      p['atom_refine_node_res'])[0]


_SCH = 512
_SUB = 128
_NSEGP = 16000


@functools.partial(jax.jit,
                   static_argnames=("s_chunk", "d", "nspace", "ncp", "cap"))
def _sc_scatter_edge(vals, idx1d, zeros, s_chunk, d, nspace, ncp, cap):
    n = vals.shape[0]
    nchunks = nspace // s_chunk
    per_sc = -(-nchunks // 2)
    per_tile = n // 16
    nvreg = per_tile // 16
    rows_cp = s_chunk // ncp
    mesh = plsc.VectorSubcoreMesh(core_axis_name="c", subcore_axis_name="s")

    @functools.partial(
        pl.kernel, mesh=mesh,
        out_type=jax.ShapeDtypeStruct((nspace, d), jnp.float32),
        compiler_params=pltpu.CompilerParams(needs_layout_passes=False),
        scratch_types=[
            pltpu.VMEM((per_tile,), jnp.int32),
            pltpu.VMEM((cap + 16,), jnp.int32),
            pltpu.VMEM((_SUB,), jnp.int32),
            pltpu.VMEM((_SUB,), jnp.int32),
            pltpu.VMEM((_SUB, d), jnp.float32),
            pltpu.VMEM_SHARED((s_chunk + 16, d), jnp.float32),
            pltpu.SemaphoreType.DMA,
        ],
    )
    def k(vals_hbm, idx_hbm, zeros_hbm, out_hbm, idxb, posb, lstage,
          pstage, rowbuf, acc, sem):
        sc = lax.axis_index("c")
        t = lax.axis_index("s")
        base_row = t * per_tile
        pltpu.sync_copy(idx_hbm.at[pl.ds(base_row, per_tile)], idxb)
        iota = lax.iota(jnp.int32, 16)
        safe = jnp.full((16,), base_row, jnp.int32)

        def pre(i2, _):
            plsc.store_scatter(posb, [i2 * 16 + iota], safe)
            return 0

        lax.fori_loop(0, (cap + 16) // 16, pre, 0)

        def chunk(mi, _):
            c = 2 * mi + sc

            @pl.when(c < nchunks)
            def _():
                lo = c * s_chunk

                @pl.when(t < ncp)
                def _():
                    pltpu.sync_copy(zeros_hbm,
                                    acc.at[pl.ds(t * rows_cp, rows_cp)])

                def scan(kk, cntv):
                    v = idxb[pl.ds(kk * 16, 16)]
                    msk = (v >= lo) & (v < lo + s_chunk)
                    mi32 = msk.astype(jnp.int32)
                    excl = plsc.cumsum(mi32) - mi32
                    dest = jnp.where(msk & (cntv + excl < cap),
                                     cntv + excl, cap + iota)
                    plsc.store_scatter(posb, [dest],
                                       base_row + kk * 16 + iota)
                    return cntv + plsc.all_reduce_population_count(msk)

                plsc.subcore_barrier()
                cntv = lax.fori_loop(0, nvreg, scan,
                                     jnp.zeros((16,), jnp.int32))
                cnt = jnp.minimum(jnp.max(cntv), cap)
                nsub = (cnt + _SUB - 1) // _SUB

                def batch(j, _):
                    for r in range(_SUB // 16):
                        lane = j * _SUB + r * 16
                        lpos = plsc.load_gather(posb, [lane + iota])
                        vi = plsc.load_gather(idxb, [lpos - base_row])
                        lid = jnp.where(lane + iota < cnt, vi - lo,
                                        s_chunk + iota)
                        lstage[pl.ds(r * 16, 16)] = lid
                        pstage[pl.ds(r * 16, 16)] = lpos
                    pltpu.async_copy(vals_hbm.at[pstage], rowbuf,
                                     sem).wait()
                    pltpu.sync_copy(rowbuf, acc.at[lstage], add=True)
                    return 0

                lax.fori_loop(0, nsub, batch, 0)
                plsc.subcore_barrier()

                @pl.when(t < ncp)
                def _():
                    pltpu.sync_copy(
                        acc.at[pl.ds(t * rows_cp, rows_cp)],
                        out_hbm.at[pl.ds(lo + t * rows_cp, rows_cp)])

                plsc.subcore_barrier()

            return 0

        lax.fori_loop(0, per_sc, chunk, 0)

    return k(vals, idx1d, zeros)


_NW = 32
_GCHUNK = 80


@functools.partial(jax.jit, static_argnames=("d",))
def _sc_gather(table, idx, d):
    b = idx.shape[0]
    per_w = b // _NW
    nchunk = per_w // _GCHUNK
    mesh = plsc.VectorSubcoreMesh(core_axis_name="c", subcore_axis_name="s")

    @functools.partial(
        pl.kernel, mesh=mesh,
        out_type=jax.ShapeDtypeStruct((b, d), jnp.float32),
        scratch_types=[
            pltpu.VMEM((_GCHUNK,), jnp.int32),
            pltpu.VMEM((_GCHUNK, d), jnp.float32),
            pltpu.VMEM((_GCHUNK,), jnp.int32),
            pltpu.VMEM((_GCHUNK, d), jnp.float32),
            pltpu.SemaphoreType.DMA,
            pltpu.SemaphoreType.DMA,
        ],
    )
    def k(table_hbm, idx_hbm, out_hbm, idx0, rows0, idx1, rows1, s0, s1):
        wid = lax.axis_index("s") * 2 + lax.axis_index("c")
        base = wid * per_w

        def issue(j, idx_v, rows_v, sem):
            pltpu.sync_copy(idx_hbm.at[pl.ds(base + j * _GCHUNK, _GCHUNK)],
                            idx_v)
            return pltpu.async_copy(table_hbm.at[idx_v], rows_v, sem)

        def drain(j, rows_v, cp):
            cp.wait()
            pltpu.sync_copy(rows_v,
                            out_hbm.at[pl.ds(base + j * _GCHUNK, _GCHUNK)])

        def body(i, _):
            j = i * 2
            cp0 = issue(j, idx0, rows0, s0)
            cp1 = issue(j + 1, idx1, rows1, s1)
            drain(j, rows0, cp0)
            drain(j + 1, rows1, cp1)
            return 0

        lax.fori_loop(0, nchunk // 2, body, 0)
        if nchunk % 2:
            drain(nchunk - 1, rows0, issue(nchunk - 1, idx0, rows0, s0))

    return k(table, idx)


def _gather(table, idx):
    return _sc_gather(table, idx, table.shape[1])


def kernel(params, node_ebd_ext, edge_ebd, h2, angle_ebd, nlist, nlist_mask,
           sw, a_nlist, a_nlist_mask, a_sw, edge_index, angle_index,
           edge_rbf, angle_rbf):
    del h2, nlist, nlist_mask, a_nlist, a_nlist_mask
    p = params
    nb, nloc, _ = node_ebd_ext.shape
    n_edge = edge_ebd.shape[0]
    n_angle = angle_ebd.shape[0]
    n2e = edge_index[0]
    next2e = edge_index[1]
    n2a = angle_index[0]
    eij2a = angle_index[1]
    eik2a = angle_index[2]
    node_flat = node_ebd_ext.reshape(-1, _NDIM)
    asw = a_sw[:, None]
    swc = sw[:, None]
    be = 2000
    bn = 1000

    ikij = jnp.concatenate([eik2a, eij2a]).astype(jnp.int32)
    nene = jnp.concatenate([n2e, next2e]).astype(jnp.int32)

    edge_dup = jnp.concatenate([edge_ebd, edge_ebd], axis=1)
    ikj = _gather(edge_dup, ikij)
    scat1, angle_1 = _d1(p, angle_ebd, ikj, asw, be)
    eij32 = eij2a.astype(jnp.int32)
    z128 = jnp.zeros((1000, 2 * _EDIM), jnp.float32)
    z64 = jnp.zeros((1000, _EDIM), jnp.float32)
    s1 = _sc_scatter_edge(scat1, eij32, z128, 8000, 2 * _EDIM,
                          n_edge, 8, 2048)

    ngg = _gather(node_flat, nene)
    scat2, e2dup = _d2(p, ngg, s1, edge_ebd, swc, be)
    n2e32 = n2e.astype(jnp.int32)
    z80 = jnp.zeros((80, 2 * _EDIM), jnp.float32)
    s2 = _sc_scatter_edge(scat2, n2e32, z80, 1280, 2 * _EDIM,
                          16640, 16, 3072)
    node_1 = _d3(p, node_flat, s2, bn)

    ikj2 = _gather(e2dup, ikij)
    ng3 = _gather(node_1, n2a.astype(jnp.int32))
    arbf = jnp.pad(angle_rbf, ((0, 0), (0, 1)))
    scat3, angle_2 = _d4(p, ng3, angle_1, ikj2, asw, arbf, be)
    s3 = _sc_scatter_edge(scat3, eij32, z128, 8000, 2 * _EDIM,
                          n_edge, 8, 2048)

    ng4 = _gather(node_1, n2e32)
    erbf = jnp.pad(edge_rbf, ((0, 0), (0, 1)))
    scat4, e4 = _d5(p, s3, e2dup, ng4, ngg, swc, erbf, be)
    s4 = _sc_scatter_edge(scat4, n2e32, z80, 1280, 2 * _EDIM,
                          16640, 16, 3072)
    node_2 = _d6(p, node_1, s4, bn)

    return node_2.reshape(nb, nloc, _NDIM), e4, angle_2


# speedup vs baseline: 3.0241x; 1.0249x over previous
import functools

import jax
import jax.numpy as jnp
import numpy as np
from jax import lax
from jax.experimental import pallas as pl
from jax.experimental.pallas import tpu as pltpu
from jax.experimental.pallas import tpu_sc as plsc

_NDIM = 128
_EDIM = 64
_ADIM = 32
_GH = 128


def _silu(x):
    return x * jax.nn.sigmoid(x)


def _dot(a, b):
    return jax.lax.dot_general(a, b, (((1,), (0,)), ((), ())),
                               preferred_element_type=jnp.float32)


def _full(shape):
    return pl.BlockSpec(shape, lambda i: (0,) * len(shape))


def _rows(b, d):
    return pl.BlockSpec((b, d), lambda i: (i, 0))


def _rows_off(b, d, off):
    return pl.BlockSpec((b, d), lambda i: (i + off, 0))


def _d1_body(ang_ref, ik_ref, ij_ref, asw_ref,
             wl_ref, gw_ref, gb_ref, uw_ref, ub_ref, ow_ref, ob_ref,
             amw_ref, amb_ref, ares_ref,
             scat_ref, angout_ref):
    ang = ang_ref[...]
    ik = ik_ref[:, 0:_EDIM]
    ij = ij_ref[:, 0:_EDIM]
    asw = asw_ref[...]
    gw = gw_ref[...]; uw = uw_ref[...]
    g = (_dot(ang, gw[0:_ADIM]) + _dot(ik, gw[_ADIM:_ADIM + _EDIM])
         + _dot(ij, gw[_ADIM + _EDIM:]) + gb_ref[...])
    u = (_dot(ang, uw[0:_ADIM]) + _dot(ik, uw[_ADIM:_ADIM + _EDIM])
         + _dot(ij, uw[_ADIM + _EDIM:]) + ub_ref[...])
    upd = _dot(_silu(g) * u, ow_ref[...]) + ob_ref[...]
    ex = jnp.exp(_dot(ang, wl_ref[...]) * asw)
    scat_ref[:, 0:_EDIM] = ex
    scat_ref[:, _EDIM:] = ex * upd * asw
    amw = amw_ref[...]
    a_upd = _silu(_dot(ang, amw[0:_ADIM]) + _dot(ik, amw[_ADIM:_ADIM + _EDIM])
                  + _dot(ij, amw[_ADIM + _EDIM:]) + amb_ref[...])
    angout_ref[...] = a_upd + ares_ref[...] * ang


def _d1(p, ang, ikj, asw, b):
    n = ang.shape[0]
    off = n // b
    m = p['line_attn_edge_mlp']
    return pl.pallas_call(
        _d1_body,
        grid=(n // b,),
        in_specs=[_rows(b, _ADIM), _rows(b, 2 * _EDIM),
                  _rows_off(b, 2 * _EDIM, off), _rows(b, 1),
                  _full((_ADIM, _EDIM)),
                  _full((_ADIM + 2 * _EDIM, _GH)), _full((1, _GH)),
                  _full((_ADIM + 2 * _EDIM, _GH)), _full((1, _GH)),
                  _full((_GH, _EDIM)), _full((1, _EDIM)),
                  _full((_ADIM + 2 * _EDIM, _ADIM)), _full((1, _ADIM)),
                  _full((1, _ADIM))],
        out_specs=[_rows(b, 2 * _EDIM), _rows(b, _ADIM)],
        out_shape=[jax.ShapeDtypeStruct((n, 2 * _EDIM), jnp.float32),
                   jax.ShapeDtypeStruct((n, _ADIM), jnp.float32)],
        compiler_params=pltpu.CompilerParams(
            dimension_semantics=("arbitrary",)),
    )(ang, ikj, ikj, asw,
      p['line_attn_weight_linear']['w'],
      m['g']['w'], m['g']['b'][None], m['u']['w'], m['u']['b'][None],
      m['o']['w'], m['o']['b'][None],
      p['line_attn_angle_mlp']['w'], p['line_attn_angle_mlp']['b'][None],
      p['line_attn_angle_res'])


def _d2_body(n1_ref, n2_ref, s1_ref, eold_ref, sw_ref,
             gw_ref, gb_ref, uw_ref, ub_ref, ow_ref, ob_ref,
             wl_ref, res1_ref, res2_ref, inv_ref,
             scat_ref, e2_ref):
    den = s1_ref[:, 0:_EDIM]
    num = s1_ref[:, _EDIM:]
    e1 = num / (den + 1e-9) * inv_ref[0, 0] + res1_ref[...] * eold_ref[...]
    n1 = n1_ref[...]
    n2 = n2_ref[...]
    sw = sw_ref[...]
    gw = gw_ref[...]; uw = uw_ref[...]
    g = (_dot(n1, gw[0:_NDIM]) + _dot(n2, gw[_NDIM:2 * _NDIM])
         + _dot(e1, gw[2 * _NDIM:]) + gb_ref[...])
    u = (_dot(n1, uw[0:_NDIM]) + _dot(n2, uw[_NDIM:2 * _NDIM])
         + _dot(e1, uw[2 * _NDIM:]) + ub_ref[...])
    aeu = _dot(_silu(g) * u, ow_ref[...]) + ob_ref[...]
    ex = jnp.exp(_dot(e1, wl_ref[...]) * sw)
    scat_ref[:, 0:_EDIM] = ex
    scat_ref[:, _EDIM:] = ex * aeu * sw
    e2 = aeu + res2_ref[...] * e1
    e2_ref[:, 0:_EDIM] = e2
    e2_ref[:, _EDIM:] = e2


def _d2(p, ngg, s1, eold, sw, b):
    n = eold.shape[0]
    off = n // b
    m = p['atom_attn_edge_mlp']
    inv = jnp.full((1, 1), 1.0 / np.sqrt(0.8), jnp.float32)
    return pl.pallas_call(
        _d2_body,
        grid=(n // b,),
        in_specs=[_rows(b, _NDIM), _rows_off(b, _NDIM, off),
                  _rows(b, 2 * _EDIM),
                  _rows(b, _EDIM), _rows(b, 1),
                  _full((2 * _NDIM + _EDIM, _GH)), _full((1, _GH)),
                  _full((2 * _NDIM + _EDIM, _GH)), _full((1, _GH)),
                  _full((_GH, _EDIM)), _full((1, _EDIM)),
                  _full((_EDIM, _EDIM)), _full((1, _EDIM)),
                  _full((1, _EDIM)), _full((1, 1))],
        out_specs=[_rows(b, 2 * _EDIM), _rows(b, 2 * _EDIM)],
        out_shape=[jax.ShapeDtypeStruct((n, 2 * _EDIM), jnp.float32),
                   jax.ShapeDtypeStruct((n, 2 * _EDIM), jnp.float32)],
        compiler_params=pltpu.CompilerParams(
            dimension_semantics=("arbitrary",)),
    )(ngg, ngg, s1, eold, sw,
      m['g']['w'], m['g']['b'][None], m['u']['w'], m['u']['b'][None],
      m['o']['w'], m['o']['b'][None],
      p['atom_attn_weight_linear']['w'],
      p['line_attn_edge_res'], p['atom_attn_edge_res'], inv)


def _d3_body(node_ref, s2_ref, gw_ref, gb_ref, uw_ref, ub_ref,
             ow_ref, ob_ref, res_ref, node1_ref):
    node = node_ref[...]
    agg = s2_ref[:, _EDIM:] / (s2_ref[:, 0:_EDIM] + 1e-9) * (1.0 / 3.2)
    gw = gw_ref[...]; uw = uw_ref[...]
    g = _dot(node, gw[0:_NDIM]) + _dot(agg, gw[_NDIM:]) + gb_ref[...]
    u = _dot(node, uw[0:_NDIM]) + _dot(agg, uw[_NDIM:]) + ub_ref[...]
    upd = _dot(_silu(g) * u, ow_ref[...]) + ob_ref[...]
    node1_ref[...] = upd + res_ref[...] * node


def _d3(p, node, s2, b):
    n = node.shape[0]
    m = p['atom_attn_node_mlp']
    return pl.pallas_call(
        _d3_body,
        grid=(n // b,),
        in_specs=[_rows(b, _NDIM), _rows(b, 2 * _EDIM),
                  _full((_NDIM + _EDIM, _GH)), _full((1, _GH)),
                  _full((_NDIM + _EDIM, _GH)), _full((1, _GH)),
                  _full((_GH, _NDIM)), _full((1, _NDIM)),
                  _full((1, _NDIM))],
        out_specs=[_rows(b, _NDIM)],
        out_shape=[jax.ShapeDtypeStruct((n, _NDIM), jnp.float32)],
        compiler_params=pltpu.CompilerParams(
            dimension_semantics=("arbitrary",)),
    )(node, s2,
      m['g']['w'], m['g']['b'][None], m['u']['w'], m['u']['b'][None],
      m['o']['w'], m['o']['b'][None], p['atom_attn_node_res'])[0]


def _d4_body(ng_ref, ang_ref, ik_ref, ij_ref, asw_ref, arbf_ref,
             gw_ref, gb_ref, uw_ref, ub_ref, ow_ref, ob_ref,
             envw_ref, effnw_ref, effnb_ref, res_ref,
             scat_ref, ang2_ref):
    ng = ng_ref[...]
    ang = ang_ref[...]
    ik = ik_ref[:, 0:_EDIM]
    ij = ij_ref[:, 0:_EDIM]
    asw = asw_ref[...]
    gw = gw_ref[...]; uw = uw_ref[...]
    c0, c1, c2 = _NDIM, _NDIM + _ADIM, _NDIM + _ADIM + _EDIM
    g = (_dot(ng, gw[0:c0]) + _dot(ang, gw[c0:c1]) + _dot(ik, gw[c1:c2])
         + _dot(ij, gw[c2:]) + gb_ref[...])
    u = (_dot(ng, uw[0:c0]) + _dot(ang, uw[c0:c1]) + _dot(ik, uw[c1:c2])
         + _dot(ij, uw[c2:]) + ub_ref[...])
    env = _dot(arbf_ref[...], envw_ref[...])
    lru = (_dot(_silu(g) * u, ow_ref[...]) + ob_ref[...]) * env
    sc3 = lru * asw
    scat_ref[:, 0:_EDIM] = sc3
    scat_ref[:, _EDIM:] = sc3
    ang2_ref[...] = _silu(_dot(lru, effnw_ref[...]) + effnb_ref[...]) \
        + res_ref[...] * ang


def _d4(p, ng, ang, ikj, asw, arbf, b):
    n = ang.shape[0]
    off = n // b
    m = p['line_refine_mlp']
    din = _NDIM + _ADIM + 2 * _EDIM
    nab = arbf.shape[1]
    return pl.pallas_call(
        _d4_body,
        grid=(n // b,),
        in_specs=[_rows(b, _NDIM), _rows(b, _ADIM), _rows(b, 2 * _EDIM),
                  _rows_off(b, 2 * _EDIM, off), _rows(b, 1), _rows(b, nab),
                  _full((din, _GH)), _full((1, _GH)),
                  _full((din, _GH)), _full((1, _GH)),
                  _full((_GH, _EDIM)), _full((1, _EDIM)),
                  _full((nab, _EDIM)),
                  _full((_EDIM, _ADIM)), _full((1, _ADIM)),
                  _full((1, _ADIM))],
        out_specs=[_rows(b, 2 * _EDIM), _rows(b, _ADIM)],
        out_shape=[jax.ShapeDtypeStruct((n, 2 * _EDIM), jnp.float32),
                   jax.ShapeDtypeStruct((n, _ADIM), jnp.float32)],
        compiler_params=pltpu.CompilerParams(
            dimension_semantics=("arbitrary",)),
    )(ng, ang, ikj, ikj, asw, arbf,
      m['g']['w'], m['g']['b'][None], m['u']['w'], m['u']['b'][None],
      m['o']['w'], m['o']['b'][None],
      jnp.pad(p['line_refine_envelope']['w'], ((0, nab - 7), (0, 0))),
      p['line_refine_edge_ffn']['w'], p['line_refine_edge_ffn']['b'][None],
      p['line_refine_angle_res'])


def _d5_body(s3_ref, e2_ref, n1_ref, n2_ref, sw_ref, erbf_ref,
             nffnw_ref, nffnb_ref, resl_ref,
             gw_ref, gb_ref, uw_ref, ub_ref, ow_ref, ob_ref,
             envw_ref, effnw_ref, effnb_ref, rese_ref,
             scat_ref, e4_ref):
    agg = s3_ref[:, 0:_EDIM] * (1.0 / 0.8)
    e3 = _silu(_dot(agg, nffnw_ref[...]) + nffnb_ref[...]) \
        + resl_ref[...] * e2_ref[:, 0:_EDIM]
    n1 = n1_ref[...]
    n2 = n2_ref[...]
    sw = sw_ref[...]
    gw = gw_ref[...]; uw = uw_ref[...]
    g = (_dot(n1, gw[0:_NDIM]) + _dot(n2, gw[_NDIM:2 * _NDIM])
         + _dot(e3, gw[2 * _NDIM:]) + gb_ref[...])
    u = (_dot(n1, uw[0:_NDIM]) + _dot(n2, uw[_NDIM:2 * _NDIM])
         + _dot(e3, uw[2 * _NDIM:]) + ub_ref[...])
    env = _dot(erbf_ref[...], envw_ref[...])
    aru = (_dot(_silu(g) * u, ow_ref[...]) + ob_ref[...]) * env
    sc4 = aru * sw
    scat_ref[:, 0:_EDIM] = sc4
    scat_ref[:, _EDIM:] = sc4
    e4_ref[...] = _silu(_dot(aru, effnw_ref[...]) + effnb_ref[...]) \
        + rese_ref[...] * e3


def _d5(p, s3, e2, n1, ngg, sw, erbf, b):
    n = e2.shape[0]
    off = n // b
    m = p['atom_refine_mlp']
    neb = erbf.shape[1]
    return pl.pallas_call(
        _d5_body,
        grid=(n // b,),
        in_specs=[_rows(b, 2 * _EDIM), _rows(b, 2 * _EDIM), _rows(b, _NDIM),
                  _rows_off(b, _NDIM, off), _rows(b, 1), _rows(b, neb),
                  _full((_EDIM, _EDIM)), _full((1, _EDIM)),
                  _full((1, _EDIM)),
                  _full((2 * _NDIM + _EDIM, _GH)), _full((1, _GH)),
                  _full((2 * _NDIM + _EDIM, _GH)), _full((1, _GH)),
                  _full((_GH, _EDIM)), _full((1, _EDIM)),
                  _full((neb, _EDIM)),
                  _full((_EDIM, _EDIM)), _full((1, _EDIM)),
                  _full((1, _EDIM))],
        out_specs=[_rows(b, 2 * _EDIM), _rows(b, _EDIM)],
        out_shape=[jax.ShapeDtypeStruct((n, 2 * _EDIM), jnp.float32),
                   jax.ShapeDtypeStruct((n, _EDIM), jnp.float32)],
        compiler_params=pltpu.CompilerParams(
            dimension_semantics=("arbitrary",)),
    )(s3, e2, n1, ngg, sw, erbf,
      p['line_refine_node_ffn']['w'], p['line_refine_node_ffn']['b'][None],
      p['line_refine_edge_res'],
      m['g']['w'], m['g']['b'][None], m['u']['w'], m['u']['b'][None],
      m['o']['w'], m['o']['b'][None],
      jnp.pad(p['atom_refine_envelope']['w'], ((0, neb - 7), (0, 0))),
      p['atom_refine_edge_ffn']['w'], p['atom_refine_edge_ffn']['b'][None],
      p['atom_refine_edge_res'])


def _d6_body(node1_ref, s4_ref, w_ref, b_ref, res_ref, node2_ref):
    agg = s4_ref[:, 0:_EDIM] * (1.0 / 3.2)
    node2_ref[...] = _silu(_dot(agg, w_ref[...]) + b_ref[...]) \
        + res_ref[...] * node1_ref[...]


def _d6(p, node1, s4, b):
    n = node1.shape[0]
    return pl.pallas_call(
        _d6_body,
        grid=(n // b,),
        in_specs=[_rows(b, _NDIM), _rows(b, 2 * _EDIM),
                  _full((_EDIM, _NDIM)), _full((1, _NDIM)),
                  _full((1, _NDIM))],
        out_specs=[_rows(b, _NDIM)],
        out_shape=[jax.ShapeDtypeStruct((n, _NDIM), jnp.float32)],
        compiler_params=pltpu.CompilerParams(
            dimension_semantics=("arbitrary",)),
    )(node1, s4,
      p['atom_refine_node_ffn']['w'], p['atom_refine_node_ffn']['b'][None],
      p['atom_refine_node_res'])[0]


_SCH = 512
_SUB = 128
_NSEGP = 16000


@functools.partial(jax.jit,
                   static_argnames=("s_chunk", "d", "nspace", "ncp", "cap"))
def _sc_scatter_edge(vals, idx1d, zeros, s_chunk, d, nspace, ncp, cap):
    n = vals.shape[0]
    nchunks = nspace // s_chunk
    per_sc = -(-nchunks // 2)
    per_tile = n // 16
    nvreg = per_tile // 16
    rows_cp = s_chunk // ncp
    mesh = plsc.VectorSubcoreMesh(core_axis_name="c", subcore_axis_name="s")

    @functools.partial(
        pl.kernel, mesh=mesh,
        out_type=jax.ShapeDtypeStruct((nspace, d), jnp.float32),
        compiler_params=pltpu.CompilerParams(needs_layout_passes=False),
        scratch_types=[
            pltpu.VMEM((per_tile,), jnp.int32),
            pltpu.VMEM((cap + 16,), jnp.int32),
            pltpu.VMEM((_SUB,), jnp.int32),
            pltpu.VMEM((_SUB,), jnp.int32),
            pltpu.VMEM((_SUB, d), jnp.float32),
            pltpu.VMEM_SHARED((s_chunk + 16, d), jnp.float32),
            pltpu.SemaphoreType.DMA,
        ],
    )
    def k(vals_hbm, idx_hbm, zeros_hbm, out_hbm, idxb, posb, lstage,
          pstage, rowbuf, acc, sem):
        sc = lax.axis_index("c")
        t = lax.axis_index("s")
        base_row = t * per_tile
        pltpu.sync_copy(idx_hbm.at[pl.ds(base_row, per_tile)], idxb)
        iota = lax.iota(jnp.int32, 16)
        safe = jnp.full((16,), base_row, jnp.int32)

        def pre(i2, _):
            plsc.store_scatter(posb, [i2 * 16 + iota], safe)
            return 0

        lax.fori_loop(0, (cap + 16) // 16, pre, 0)

        def chunk(mi, _):
            c = 2 * mi + sc

            @pl.when(c < nchunks)
            def _():
                lo = c * s_chunk

                @pl.when(t < ncp)
                def _():
                    pltpu.sync_copy(zeros_hbm,
                                    acc.at[pl.ds(t * rows_cp, rows_cp)])

                def scan(kk, cntv):
                    v = idxb[pl.ds(kk * 16, 16)]
                    msk = (v >= lo) & (v < lo + s_chunk)
                    mi32 = msk.astype(jnp.int32)
                    excl = plsc.cumsum(mi32) - mi32
                    dest = jnp.where(msk & (cntv + excl < cap),
                                     cntv + excl, cap + iota)
                    plsc.store_scatter(posb, [dest],
                                       base_row + kk * 16 + iota)
                    return cntv + plsc.all_reduce_population_count(msk)

                plsc.subcore_barrier()
                cntv = lax.fori_loop(0, nvreg, scan,
                                     jnp.zeros((16,), jnp.int32))
                cnt = jnp.minimum(jnp.max(cntv), cap)
                nsub = (cnt + _SUB - 1) // _SUB

                def batch(j, _):
                    for r in range(_SUB // 16):
                        lane = j * _SUB + r * 16
                        lpos = plsc.load_gather(posb, [lane + iota])
                        vi = plsc.load_gather(idxb, [lpos - base_row])
                        lid = jnp.where(lane + iota < cnt, vi - lo,
                                        s_chunk + iota)
                        lstage[pl.ds(r * 16, 16)] = lid
                        pstage[pl.ds(r * 16, 16)] = lpos
                    pltpu.async_copy(vals_hbm.at[pstage], rowbuf,
                                     sem).wait()
                    pltpu.sync_copy(rowbuf, acc.at[lstage], add=True)
                    return 0

                lax.fori_loop(0, nsub, batch, 0)
                plsc.subcore_barrier()

                @pl.when(t < ncp)
                def _():
                    pltpu.sync_copy(
                        acc.at[pl.ds(t * rows_cp, rows_cp)],
                        out_hbm.at[pl.ds(lo + t * rows_cp, rows_cp)])

                plsc.subcore_barrier()

            return 0

        lax.fori_loop(0, per_sc, chunk, 0)

    return k(vals, idx1d, zeros)


_NW = 32
_GCHUNK = 80


@functools.partial(jax.jit, static_argnames=("d",))
def _sc_gather(table, idx, d):
    b = idx.shape[0]
    per_w = b // _NW
    nchunk = per_w // _GCHUNK
    ring = 5
    groups = nchunk // ring
    assert groups * ring == nchunk
    mesh = plsc.VectorSubcoreMesh(core_axis_name="c", subcore_axis_name="s")

    scratch = []
    for _ in range(ring):
        scratch += [pltpu.VMEM((_GCHUNK,), jnp.int32),
                    pltpu.VMEM((_GCHUNK, d), jnp.float32),
                    pltpu.SemaphoreType.DMA]

    @functools.partial(
        pl.kernel, mesh=mesh,
        out_type=jax.ShapeDtypeStruct((b, d), jnp.float32),
        scratch_types=scratch,
    )
    def k(table_hbm, idx_hbm, out_hbm, *bufs):
        wid = lax.axis_index("s") * 2 + lax.axis_index("c")
        base = wid * per_w
        slots = [(bufs[3 * r], bufs[3 * r + 1], bufs[3 * r + 2])
                 for r in range(ring)]

        def issue(j, sl):
            idx_v, rows_v, sem = sl
            pltpu.sync_copy(idx_hbm.at[pl.ds(base + j * _GCHUNK, _GCHUNK)],
                            idx_v)
            pltpu.async_copy(table_hbm.at[idx_v], rows_v, sem)

        def drain(j, sl):
            idx_v, rows_v, sem = sl
            pltpu.make_async_copy(table_hbm.at[idx_v], rows_v, sem).wait()
            pltpu.sync_copy(rows_v,
                            out_hbm.at[pl.ds(base + j * _GCHUNK, _GCHUNK)])

        for r in range(ring):
            issue(r, slots[r])

        def body(i, _):
            for r in range(ring):
                j = i * ring + r
                drain(j, slots[r])
                issue(j + ring, slots[r])
            return 0

        lax.fori_loop(0, groups - 1, body, 0)
        for r in range(ring):
            drain((groups - 1) * ring + r, slots[r])

    return k(table, idx)


def _gather(table, idx):
    return _sc_gather(table, idx, table.shape[1])


def kernel(params, node_ebd_ext, edge_ebd, h2, angle_ebd, nlist, nlist_mask,
           sw, a_nlist, a_nlist_mask, a_sw, edge_index, angle_index,
           edge_rbf, angle_rbf):
    del h2, nlist, nlist_mask, a_nlist, a_nlist_mask
    p = params
    nb, nloc, _ = node_ebd_ext.shape
    n_edge = edge_ebd.shape[0]
    n_angle = angle_ebd.shape[0]
    n2e = edge_index[0]
    next2e = edge_index[1]
    n2a = angle_index[0]
    eij2a = angle_index[1]
    eik2a = angle_index[2]
    node_flat = node_ebd_ext.reshape(-1, _NDIM)
    asw = a_sw[:, None]
    swc = sw[:, None]
    be = 2000
    bn = 1000

    ikij = jnp.concatenate([eik2a, eij2a]).astype(jnp.int32)
    nene = jnp.concatenate([n2e, next2e]).astype(jnp.int32)

    edge_dup = jnp.concatenate([edge_ebd, edge_ebd], axis=1)
    ikj = _gather(edge_dup, ikij)
    scat1, angle_1 = _d1(p, angle_ebd, ikj, asw, be)
    eij32 = eij2a.astype(jnp.int32)
    z128 = jnp.zeros((1000, 2 * _EDIM), jnp.float32)
    z64 = jnp.zeros((1000, _EDIM), jnp.float32)
    z504 = jnp.zeros((504, 2 * _EDIM), jnp.float32)
    s1 = _sc_scatter_edge(scat1, eij32, z504, 8064, 2 * _EDIM,
                          322560, 16, 2048)

    ngg = _gather(node_flat, nene)
    scat2, e2dup = _d2(p, ngg, s1, edge_ebd, swc, be)
    n2e32 = n2e.astype(jnp.int32)
    z80 = jnp.zeros((80, 2 * _EDIM), jnp.float32)
    s2 = _sc_scatter_edge(scat2, n2e32, z80, 1280, 2 * _EDIM,
                          16640, 16, 3072)
    node_1 = _d3(p, node_flat, s2, bn)

    ikj2 = _gather(e2dup, ikij)
    ng3 = _gather(node_1, n2a.astype(jnp.int32))
    arbf = jnp.pad(angle_rbf, ((0, 0), (0, 1)))
    scat3, angle_2 = _d4(p, ng3, angle_1, ikj2, asw, arbf, be)
    s3 = _sc_scatter_edge(scat3, eij32, z504, 8064, 2 * _EDIM,
                          322560, 16, 2048)

    ng4 = _gather(node_1, n2e32)
    erbf = jnp.pad(edge_rbf, ((0, 0), (0, 1)))
    scat4, e4 = _d5(p, s3, e2dup, ng4, ngg, swc, erbf, be)
    s4 = _sc_scatter_edge(scat4, n2e32, z80, 1280, 2 * _EDIM,
                          16640, 16, 3072)
    node_2 = _d6(p, node_1, s4, bn)

    return node_2.reshape(nb, nloc, _NDIM), e4, angle_2
